# Initial kernel scaffold; baseline (speedup 1.0000x reference)
import jax
import jax.numpy as jnp
from jax.experimental import pallas as pl


def kernel(x, edge_index, edge_attr, edge_gate_type, batch, global_features, threshold_class, W0, b0, g0, be0, mpWmsg, mpbmsg, mpWedge, mpGemb, mpWout, mpbout, mpWself, Wg, bg, gg, bgg, Temb, Wt, bt, W1, b1, g1, bb1, W2, b2, Wr1, br1, Wr2, br2):
    raise NotImplementedError("write your pallas kernel here")


# speedup vs baseline: 1.4495x; 1.4495x over previous
import functools

import jax
import jax.numpy as jnp
from jax import lax
from jax.experimental import pallas as pl
from jax.experimental.pallas import tpu as pltpu
from jax.experimental.pallas import tpu_sc as plsc

N = 100000
E = 1600000
NF = 16
EF = 4
GF = 36
H = 64
L = 4
G = 8
B = 64
T = 9

FG = 4
FW = 16
NBLK = 1000
EBLK = 2000
NTILES = 16
EPT = E // NTILES
W = 400
NWIN = EPT // W
NPAD = 100096
NPT = NPAD // NTILES
ZCH = 368
NCH = NPT // ZCH


def _ln(v, g, b):
    m = jnp.mean(v, axis=-1, keepdims=True)
    var = jnp.mean((v - m) * (v - m), axis=-1, keepdims=True)
    return (v - m) * lax.rsqrt(var + 1e-5) * g + b


def _init_body(x_ref, W0_ref, b0_ref, g0_ref, be0_ref, Wm_ref, h_ref, hm_ref):
    h = jnp.maximum(jnp.dot(x_ref[...], W0_ref[...],
                            preferred_element_type=jnp.float32) + b0_ref[...], 0.0)
    h = _ln(h, g0_ref[...], be0_ref[...])
    h_ref[...] = h
    hm = jnp.dot(h, Wm_ref[...], preferred_element_type=jnp.float32)
    for fg in range(FG):
        hm_ref[fg] = hm[:, fg * FW:(fg + 1) * FW]


_init_call = pl.pallas_call(
    _init_body,
    grid=(N // NBLK,),
    in_specs=[
        pl.BlockSpec((NBLK, NF), lambda i: (i, 0)),
        pl.BlockSpec((NF, H), lambda i: (0, 0)),
        pl.BlockSpec((1, H), lambda i: (0, 0)),
        pl.BlockSpec((1, H), lambda i: (0, 0)),
        pl.BlockSpec((1, H), lambda i: (0, 0)),
        pl.BlockSpec((H, H), lambda i: (0, 0)),
    ],
    out_specs=[
        pl.BlockSpec((NBLK, H), lambda i: (i, 0)),
        pl.BlockSpec((FG, NBLK, FW), lambda i: (0, i, 0)),
    ],
    out_shape=[
        jax.ShapeDtypeStruct((N, H), jnp.float32),
        jax.ShapeDtypeStruct((FG, N, FW), jnp.float32),
    ],
)


def _econst_body(ea_ref, gate_ref, We_ref, Ge_ref, bm_ref, out_ref):
    gate = gate_ref[0]
    oh = (gate == lax.broadcasted_iota(jnp.int32, (EBLK, G), 1)).astype(jnp.float32)
    m = (jnp.dot(ea_ref[...], We_ref[0], preferred_element_type=jnp.float32)
         + jnp.dot(oh, Ge_ref[0], preferred_element_type=jnp.float32)
         + bm_ref[0])
    for fg in range(FG):
        out_ref[0, fg] = m[:, fg * FW:(fg + 1) * FW]


_econst_call = pl.pallas_call(
    _econst_body,
    grid=(L, E // EBLK),
    in_specs=[
        pl.BlockSpec((EBLK, EF), lambda l, e: (e, 0)),
        pl.BlockSpec((1, EBLK, 1), lambda l, e: (e, 0, 0)),
        pl.BlockSpec((1, EF, H), lambda l, e: (l, 0, 0)),
        pl.BlockSpec((1, G, H), lambda l, e: (l, 0, 0)),
        pl.BlockSpec((1, 1, H), lambda l, e: (l, 0, 0)),
    ],
    out_specs=pl.BlockSpec((1, FG, EBLK, FW), lambda l, e: (l, 0, e, 0)),
    out_shape=jax.ShapeDtypeStruct((L, FG, E, FW), jnp.float32),
)


def _update_body(h_ref, agg_ref, Wout_ref, Wself_ref, bout_ref, Wm_ref,
                 hout_ref, hm_ref):
    h = h_ref[...]
    acc = jnp.dot(h, Wself_ref[...], preferred_element_type=jnp.float32) + bout_ref[...]
    for fg in range(FG):
        acc = acc + jnp.dot(agg_ref[fg], Wout_ref[...][fg * FW:(fg + 1) * FW, :],
                            preferred_element_type=jnp.float32)
    h2 = h + jnp.maximum(acc, 0.0)
    hout_ref[...] = h2
    hm = jnp.dot(h2, Wm_ref[...], preferred_element_type=jnp.float32)
    for fg in range(FG):
        hm_ref[fg] = hm[:, fg * FW:(fg + 1) * FW]


_update_call = pl.pallas_call(
    _update_body,
    grid=(N // NBLK,),
    in_specs=[
        pl.BlockSpec((NBLK, H), lambda i: (i, 0)),
        pl.BlockSpec((FG, NBLK, FW), lambda i: (0, i, 0)),
        pl.BlockSpec((H, H), lambda i: (0, 0)),
        pl.BlockSpec((H, H), lambda i: (0, 0)),
        pl.BlockSpec((1, H), lambda i: (0, 0)),
        pl.BlockSpec((H, H), lambda i: (0, 0)),
    ],
    out_specs=[
        pl.BlockSpec((NBLK, H), lambda i: (i, 0)),
        pl.BlockSpec((FG, NBLK, FW), lambda i: (0, i, 0)),
    ],
    out_shape=[
        jax.ShapeDtypeStruct((N, H), jnp.float32),
        jax.ShapeDtypeStruct((FG, N, FW), jnp.float32),
    ],
)


def _make_sc(layer):
    mesh = plsc.VectorSubcoreMesh(core_axis_name="c", subcore_axis_name="s")

    @functools.partial(
        pl.kernel,
        mesh=mesh,
        out_type=jax.ShapeDtypeStruct((FG * NPAD, FW), jnp.float32),
        compiler_params=pltpu.CompilerParams(use_tc_tiling_on_sc=False),
        scratch_types=[
            pltpu.VMEM((W,), jnp.int32),
            pltpu.VMEM((W,), jnp.int32),
            pltpu.VMEM((W,), jnp.int32),
            pltpu.VMEM((W, FW), jnp.float32),
            pltpu.VMEM((W, FW), jnp.float32),
            pltpu.VMEM_SHARED((NPAD, FW), jnp.float32),
            pltpu.SemaphoreType.DMA,
        ],
    )
    def sc_kernel(hm_hbm, e_hbm, src_hbm, dst_hbm, zero_hbm, out_hbm,
                  srcv, dstv, idxv, elv, hrv, aggs, gsem):
        cid = lax.axis_index("c")
        sid = lax.axis_index("s")
        for fgi in range(2):
            fg = fgi * 2 + cid
            pltpu.sync_copy(zero_hbm, elv.at[pl.ds(0, ZCH)])

            def zbody(k, _):
                off = pl.multiple_of(sid * NPT + k * ZCH, 8)
                pltpu.sync_copy(elv.at[pl.ds(0, ZCH)], aggs.at[pl.ds(off, ZCH)])
                return 0
            lax.fori_loop(0, NCH, zbody, 0)
            plsc.subcore_barrier()

            tile_lo = sid * EPT
            eoff = (layer * FG) * E + fg * E

            def wbody(w, _):
                base = pl.multiple_of(tile_lo + w * W, 8)
                ebase = pl.multiple_of(eoff + tile_lo + w * W, 8)
                pltpu.sync_copy(src_hbm.at[pl.ds(base, W)], srcv)
                pltpu.sync_copy(dst_hbm.at[pl.ds(base, W)], dstv)
                pltpu.sync_copy(e_hbm.at[pl.ds(ebase, W)], elv)
                fgN = fg * N

                def ib(j, _):
                    idxv[pl.ds(j * 16, 16)] = srcv[pl.ds(j * 16, 16)] + fgN
                    return 0
                lax.fori_loop(0, W // 16, ib, 0)
                pltpu.async_copy(hm_hbm.at[idxv], hrv, gsem).wait()

                def mb(j, _):
                    b16 = j * 16
                    for t in range(16):
                        elv[b16 + t] = jnp.maximum(elv[b16 + t] + hrv[b16 + t], 0.0)
                    return 0
                lax.fori_loop(0, W // 16, mb, 0)
                pltpu.sync_copy(elv, aggs.at[dstv], add=True)
                return 0
            lax.fori_loop(0, NWIN, wbody, 0)
            plsc.subcore_barrier()

            def dbody(k, _):
                off = pl.multiple_of(sid * NPT + k * ZCH, 8)
                oout = pl.multiple_of(fg * NPAD + sid * NPT + k * ZCH, 8)
                pltpu.sync_copy(aggs.at[pl.ds(off, ZCH)], elv.at[pl.ds(0, ZCH)])
                pltpu.sync_copy(elv.at[pl.ds(0, ZCH)], out_hbm.at[pl.ds(oout, ZCH)])
                return 0
            lax.fori_loop(0, NCH, dbody, 0)
            plsc.subcore_barrier()

    return sc_kernel


_SC_CALLS = [_make_sc(l) for l in range(L)]


def _pool_body(h_ref, batch_ref, sum_ref, max_ref, cnt_ref):
    i = pl.program_id(0)

    @pl.when(i == 0)
    def _():
        sum_ref[...] = jnp.zeros_like(sum_ref)
        cnt_ref[...] = jnp.zeros_like(cnt_ref)
        max_ref[...] = jnp.full_like(max_ref, -jnp.inf)

    h = h_ref[...]
    bid = batch_ref[0]
    oh = (bid == lax.broadcasted_iota(jnp.int32, (NBLK, B), 1)).astype(jnp.float32)
    dn = (((0,), (0,)), ((), ()))
    sum_ref[...] += lax.dot_general(oh, h, dn, preferred_element_type=jnp.float32)
    cnt_ref[...] += lax.dot_general(oh, jnp.ones_like(h), dn,
                                    preferred_element_type=jnp.float32)
    lo = bid[0, 0]
    hi = bid[NBLK - 1, 0]
    for b in range(B):
        @pl.when((b >= lo) & (b <= hi))
        def _():
            mb = jnp.max(jnp.where(bid == b, h, -jnp.inf), axis=0)
            max_ref[b:b + 1, :] = jnp.maximum(max_ref[b:b + 1, :], mb[None, :])


_pool_call = pl.pallas_call(
    _pool_body,
    grid=(N // NBLK,),
    in_specs=[
        pl.BlockSpec((NBLK, H), lambda i: (i, 0)),
        pl.BlockSpec((1, NBLK, 1), lambda i: (i, 0, 0)),
    ],
    out_specs=[
        pl.BlockSpec((B, H), lambda i: (0, 0)),
        pl.BlockSpec((B, H), lambda i: (0, 0)),
        pl.BlockSpec((B, H), lambda i: (0, 0)),
    ],
    out_shape=[
        jax.ShapeDtypeStruct((B, H), jnp.float32),
        jax.ShapeDtypeStruct((B, H), jnp.float32),
        jax.ShapeDtypeStruct((B, H), jnp.float32),
    ],
)


def _head_body(sum_ref, max_ref, cnt_ref, gf_ref, tcls_ref, Temb_ref,
               Wg_ref, bg_ref, gg_ref, bgg_ref, Wt_ref, bt_ref,
               W1_ref, b1_ref, g1_ref, bb1_ref, W2_ref, b2_ref,
               Wr1_ref, br1_ref, Wr2_ref, br2_ref, out_ref):
    cnt = cnt_ref[...]
    hsum = sum_ref[...]
    hmean = hsum / jnp.maximum(cnt, 1.0)
    hmax = jnp.where(cnt > 0, max_ref[...], 0.0)
    g = _ln(jnp.maximum(jnp.dot(gf_ref[...], Wg_ref[...],
                                preferred_element_type=jnp.float32) + bg_ref[...], 0.0),
            gg_ref[...], bgg_ref[...])
    toh = (tcls_ref[...] == lax.broadcasted_iota(jnp.int32, (B, 16), 1)).astype(jnp.float32)
    te = jnp.dot(toh, Temb_ref[...], preferred_element_type=jnp.float32)
    temb = jnp.maximum(jnp.dot(te, Wt_ref[...],
                               preferred_element_type=jnp.float32) + bt_ref[...], 0.0)
    c = jnp.concatenate([hmean, hmax, hsum, g, temb], axis=-1)
    c = _ln(jnp.maximum(jnp.dot(c, W1_ref[...],
                                preferred_element_type=jnp.float32) + b1_ref[...], 0.0),
            g1_ref[...], bb1_ref[...])
    c = jnp.maximum(jnp.dot(c, W2_ref[...],
                            preferred_element_type=jnp.float32) + b2_ref[...], 0.0)
    r = jnp.maximum(jnp.dot(c, Wr1_ref[...],
                            preferred_element_type=jnp.float32) + br1_ref[...], 0.0)
    out = jnp.dot(r, Wr2_ref[...], preferred_element_type=jnp.float32) + br2_ref[...]
    out_ref[...] = jnp.broadcast_to(out, (B, 128))


_head_call = pl.pallas_call(
    _head_body,
    out_shape=jax.ShapeDtypeStruct((B, 128), jnp.float32),
)


def kernel(x, edge_index, edge_attr, edge_gate_type, batch, global_features,
           threshold_class, W0, b0, g0, be0, mpWmsg, mpbmsg, mpWedge, mpGemb,
           mpWout, mpbout, mpWself, Wg, bg, gg, bgg, Temb, Wt, bt, W1, b1,
           g1, bb1, W2, b2, Wr1, br1, Wr2, br2):
    src = edge_index[0]
    dst = edge_index[1]
    gate3 = edge_gate_type.reshape(E // EBLK, EBLK, 1)
    batch3 = batch.reshape(N // NBLK, NBLK, 1)
    r1 = lambda a: a.reshape(1, -1)
    zero_chunk = jnp.zeros((ZCH, FW), jnp.float32)
    Temb_pad = jnp.pad(Temb, ((0, 16 - T), (0, 0)))

    h, hm = _init_call(x, W0, r1(b0), r1(g0), r1(be0), mpWmsg[0])
    e_all = _econst_call(edge_attr, gate3, mpWedge, mpGemb,
                         mpbmsg.reshape(L, 1, H))
    e_flat = e_all.reshape(L * FG * E, FW)
    for l in range(L):
        agg = _SC_CALLS[l](hm.reshape(FG * N, FW), e_flat, src, dst, zero_chunk)
        h, hm = _update_call(h, agg.reshape(FG, NPAD, FW), mpWout[l], mpWself[l],
                             r1(mpbout[l]), mpWmsg[(l + 1) % L])
    hsum, hmax, hcnt = _pool_call(h, batch3)
    out = _head_call(hsum, hmax, hcnt, global_features,
                     threshold_class.reshape(B, 1).astype(jnp.int32), Temb_pad,
                     Wg, r1(bg), r1(gg), r1(bgg), Wt, r1(bt),
                     W1, r1(b1), r1(g1), r1(bb1), W2, r1(b2),
                     Wr1, r1(br1), Wr2, r1(br2))
    return out[:, 0]


# speedup vs baseline: 1.7435x; 1.2028x over previous
import functools

import jax
import jax.numpy as jnp
from jax import lax
from jax.experimental import pallas as pl
from jax.experimental.pallas import tpu as pltpu
from jax.experimental.pallas import tpu_sc as plsc

N = 100000
E = 1600000
NF = 16
EF = 4
GF = 36
H = 64
L = 4
G = 8
B = 64
T = 9

FG = 4
FW = 16
NBLK = 1000
EBLK = 2000
NTILES = 16
EPT = E // NTILES
W = 400
NWIN = EPT // W
NPAD = 100096
NPT = NPAD // NTILES
ZCH = 368
NCH = NPT // ZCH


def _ln(v, g, b):
    m = jnp.mean(v, axis=-1, keepdims=True)
    var = jnp.mean((v - m) * (v - m), axis=-1, keepdims=True)
    return (v - m) * lax.rsqrt(var + 1e-5) * g + b


def _init_body(x_ref, W0_ref, b0_ref, g0_ref, be0_ref, Wm_ref, h_ref, hm_ref):
    h = jnp.maximum(jnp.dot(x_ref[...], W0_ref[...],
                            preferred_element_type=jnp.float32) + b0_ref[...], 0.0)
    h = _ln(h, g0_ref[...], be0_ref[...])
    h_ref[...] = h
    hm = jnp.dot(h, Wm_ref[...], preferred_element_type=jnp.float32)
    for fg in range(FG):
        hm_ref[fg] = hm[:, fg * FW:(fg + 1) * FW]


_init_call = pl.pallas_call(
    _init_body,
    grid=(N // NBLK,),
    in_specs=[
        pl.BlockSpec((NBLK, NF), lambda i: (i, 0)),
        pl.BlockSpec((NF, H), lambda i: (0, 0)),
        pl.BlockSpec((1, H), lambda i: (0, 0)),
        pl.BlockSpec((1, H), lambda i: (0, 0)),
        pl.BlockSpec((1, H), lambda i: (0, 0)),
        pl.BlockSpec((H, H), lambda i: (0, 0)),
    ],
    out_specs=[
        pl.BlockSpec((NBLK, H), lambda i: (i, 0)),
        pl.BlockSpec((FG, NBLK, FW), lambda i: (0, i, 0)),
    ],
    out_shape=[
        jax.ShapeDtypeStruct((N, H), jnp.float32),
        jax.ShapeDtypeStruct((FG, N, FW), jnp.float32),
    ],
)


def _econst_body(ea_ref, gate_ref, We_ref, Ge_ref, bm_ref, out_ref):
    gate = gate_ref[0]
    oh = (gate == lax.broadcasted_iota(jnp.int32, (EBLK, G), 1)).astype(jnp.float32)
    m = (jnp.dot(ea_ref[...], We_ref[0], preferred_element_type=jnp.float32)
         + jnp.dot(oh, Ge_ref[0], preferred_element_type=jnp.float32)
         + bm_ref[0])
    for fg in range(FG):
        out_ref[0, fg] = m[:, fg * FW:(fg + 1) * FW]


_econst_call = pl.pallas_call(
    _econst_body,
    grid=(L, E // EBLK),
    in_specs=[
        pl.BlockSpec((EBLK, EF), lambda l, e: (e, 0)),
        pl.BlockSpec((1, EBLK, 1), lambda l, e: (e, 0, 0)),
        pl.BlockSpec((1, EF, H), lambda l, e: (l, 0, 0)),
        pl.BlockSpec((1, G, H), lambda l, e: (l, 0, 0)),
        pl.BlockSpec((1, 1, H), lambda l, e: (l, 0, 0)),
    ],
    out_specs=pl.BlockSpec((1, FG, EBLK, FW), lambda l, e: (l, 0, e, 0)),
    out_shape=jax.ShapeDtypeStruct((L, FG, E, FW), jnp.float32),
)


def _update_body(h_ref, agg_ref, Wout_ref, Wself_ref, bout_ref, Wm_ref,
                 hout_ref, hm_ref):
    h = h_ref[...]
    acc = jnp.dot(h, Wself_ref[...], preferred_element_type=jnp.float32) + bout_ref[...]
    for fg in range(FG):
        acc = acc + jnp.dot(agg_ref[fg], Wout_ref[...][fg * FW:(fg + 1) * FW, :],
                            preferred_element_type=jnp.float32)
    h2 = h + jnp.maximum(acc, 0.0)
    hout_ref[...] = h2
    hm = jnp.dot(h2, Wm_ref[...], preferred_element_type=jnp.float32)
    for fg in range(FG):
        hm_ref[fg] = hm[:, fg * FW:(fg + 1) * FW]


_update_call = pl.pallas_call(
    _update_body,
    grid=(N // NBLK,),
    in_specs=[
        pl.BlockSpec((NBLK, H), lambda i: (i, 0)),
        pl.BlockSpec((FG, NBLK, FW), lambda i: (0, i, 0)),
        pl.BlockSpec((H, H), lambda i: (0, 0)),
        pl.BlockSpec((H, H), lambda i: (0, 0)),
        pl.BlockSpec((1, H), lambda i: (0, 0)),
        pl.BlockSpec((H, H), lambda i: (0, 0)),
    ],
    out_specs=[
        pl.BlockSpec((NBLK, H), lambda i: (i, 0)),
        pl.BlockSpec((FG, NBLK, FW), lambda i: (0, i, 0)),
    ],
    out_shape=[
        jax.ShapeDtypeStruct((N, H), jnp.float32),
        jax.ShapeDtypeStruct((FG, N, FW), jnp.float32),
    ],
)


def _make_sc(layer):
    mesh = plsc.VectorSubcoreMesh(core_axis_name="c", subcore_axis_name="s")

    @functools.partial(
        pl.kernel,
        mesh=mesh,
        out_type=jax.ShapeDtypeStruct((FG * NPAD, FW), jnp.float32),
        compiler_params=pltpu.CompilerParams(use_tc_tiling_on_sc=False),
        scratch_types=[
            pltpu.VMEM((W,), jnp.int32),
            pltpu.VMEM((W,), jnp.int32),
            pltpu.VMEM((W, FW), jnp.float32),
            pltpu.VMEM((W, FW), jnp.float32),
            pltpu.VMEM((W,), jnp.int32),
            pltpu.VMEM((W,), jnp.int32),
            pltpu.VMEM((W, FW), jnp.float32),
            pltpu.VMEM((W, FW), jnp.float32),
            pltpu.VMEM_SHARED((NPAD, FW), jnp.float32),
            pltpu.SemaphoreType.DMA,
            pltpu.SemaphoreType.DMA,
            pltpu.SemaphoreType.DMA,
            pltpu.SemaphoreType.DMA,
            pltpu.SemaphoreType.DMA,
            pltpu.SemaphoreType.DMA,
        ],
    )
    def sc_kernel(hm_hbm, e_hbm, src_hbm, dst_hbm, zero_hbm, out_hbm,
                  srcv0, dstv0, elv0, hrv0, srcv1, dstv1, elv1, hrv1, aggs,
                  insem0, insem1, gsem0, gsem1, ssem0, ssem1):
        cid = lax.axis_index("c")
        sid = lax.axis_index("s")
        slots = ((srcv0, dstv0, elv0, hrv0, insem0, gsem0, ssem0),
                 (srcv1, dstv1, elv1, hrv1, insem1, gsem1, ssem1))
        for fgi in range(2):
            fg = fgi * 2 + cid
            pltpu.sync_copy(zero_hbm, elv0.at[pl.ds(0, ZCH)])

            def zbody(k, _):
                off = pl.multiple_of(sid * NPT + k * ZCH, 8)
                pltpu.sync_copy(elv0.at[pl.ds(0, ZCH)], aggs.at[pl.ds(off, ZCH)])
                return 0
            lax.fori_loop(0, NCH, zbody, 0)
            plsc.subcore_barrier()

            tile_lo = sid * EPT
            eoff = (layer * FG) * E + fg * E
            fgN = fg * N

            def _in_args(w, s):
                base = pl.multiple_of(tile_lo + w * W, 8)
                ebase = pl.multiple_of(eoff + tile_lo + w * W, 8)
                return (
                    (src_hbm.at[pl.ds(base, W)], s[0], s[4]),
                    (dst_hbm.at[pl.ds(base, W)], s[1], s[4]),
                    (e_hbm.at[pl.ds(ebase, W)], s[2], s[4]),
                )

            def inputs_start(w, s):
                for a in _in_args(w, s):
                    pltpu.async_copy(*a)

            def inputs_wait(w, s):
                for a in _in_args(w, s):
                    pltpu.make_async_copy(*a).wait()

            def idx_compute(s):
                def ib(j, _):
                    s[0][pl.ds(j * 16, 16)] = s[0][pl.ds(j * 16, 16)] + fgN
                    return 0
                lax.fori_loop(0, W // 16, ib, 0)

            def gather_start(s):
                pltpu.async_copy(hm_hbm.at[s[0]], s[3], s[5])

            def gather_wait(s):
                pltpu.make_async_copy(hm_hbm.at[s[0]], s[3], s[5]).wait()

            def msg_compute(s):
                ev, hv = s[2], s[3]

                def mb(j, _):
                    b16 = j * 16
                    for t in range(16):
                        hv[b16 + t] = jnp.maximum(ev[b16 + t] + hv[b16 + t], 0.0)
                    return 0
                lax.fori_loop(0, W // 16, mb, 0)

            def scatter_start(s):
                pltpu.async_copy(s[3], aggs.at[s[1]], s[6], add=True)

            def scatter_wait(s):
                pltpu.make_async_copy(s[3], aggs.at[s[1]], s[6]).wait()

            inputs_start(0, slots[0])
            inputs_wait(0, slots[0])
            idx_compute(slots[0])
            gather_start(slots[0])
            inputs_start(1, slots[1])

            def pair(k, _):
                w1 = 2 * k + 1
                w2 = 2 * k + 2
                w3 = 2 * k + 3
                s0, s1 = slots
                gather_wait(s0)
                msg_compute(s0)
                scatter_start(s0)
                inputs_wait(w1, s1)
                idx_compute(s1)
                gather_start(s1)
                scatter_wait(s0)

                @pl.when(w2 < NWIN)
                def _():
                    inputs_start(w2, s0)
                gather_wait(s1)
                msg_compute(s1)
                scatter_start(s1)

                @pl.when(w2 < NWIN)
                def _():
                    inputs_wait(w2, s0)
                    idx_compute(s0)
                    gather_start(s0)
                scatter_wait(s1)

                @pl.when(w3 < NWIN)
                def _():
                    inputs_start(w3, s1)
                return 0
            lax.fori_loop(0, NWIN // 2, pair, 0)
            plsc.subcore_barrier()

            def dbody(k, _):
                off = pl.multiple_of(sid * NPT + k * ZCH, 8)
                oout = pl.multiple_of(fg * NPAD + sid * NPT + k * ZCH, 8)
                pltpu.sync_copy(aggs.at[pl.ds(off, ZCH)], elv0.at[pl.ds(0, ZCH)])
                pltpu.sync_copy(elv0.at[pl.ds(0, ZCH)], out_hbm.at[pl.ds(oout, ZCH)])
                return 0
            lax.fori_loop(0, NCH, dbody, 0)
            plsc.subcore_barrier()

    return sc_kernel


_SC_CALLS = [_make_sc(l) for l in range(L)]


def _pool_body(h_ref, batch_ref, sum_ref, max_ref, cnt_ref):
    i = pl.program_id(0)

    @pl.when(i == 0)
    def _():
        sum_ref[...] = jnp.zeros_like(sum_ref)
        cnt_ref[...] = jnp.zeros_like(cnt_ref)
        max_ref[...] = jnp.full_like(max_ref, -jnp.inf)

    h = h_ref[...]
    bid = batch_ref[0]
    oh = (bid == lax.broadcasted_iota(jnp.int32, (NBLK, B), 1)).astype(jnp.float32)
    dn = (((0,), (0,)), ((), ()))
    sum_ref[...] += lax.dot_general(oh, h, dn, preferred_element_type=jnp.float32)
    cnt_ref[...] += lax.dot_general(oh, jnp.ones_like(h), dn,
                                    preferred_element_type=jnp.float32)
    lo = bid[0, 0]
    hi = bid[NBLK - 1, 0]
    for b in range(B):
        @pl.when((b >= lo) & (b <= hi))
        def _():
            mb = jnp.max(jnp.where(bid == b, h, -jnp.inf), axis=0)
            max_ref[b:b + 1, :] = jnp.maximum(max_ref[b:b + 1, :], mb[None, :])


_pool_call = pl.pallas_call(
    _pool_body,
    grid=(N // NBLK,),
    in_specs=[
        pl.BlockSpec((NBLK, H), lambda i: (i, 0)),
        pl.BlockSpec((1, NBLK, 1), lambda i: (i, 0, 0)),
    ],
    out_specs=[
        pl.BlockSpec((B, H), lambda i: (0, 0)),
        pl.BlockSpec((B, H), lambda i: (0, 0)),
        pl.BlockSpec((B, H), lambda i: (0, 0)),
    ],
    out_shape=[
        jax.ShapeDtypeStruct((B, H), jnp.float32),
        jax.ShapeDtypeStruct((B, H), jnp.float32),
        jax.ShapeDtypeStruct((B, H), jnp.float32),
    ],
)


def _head_body(sum_ref, max_ref, cnt_ref, gf_ref, tcls_ref, Temb_ref,
               Wg_ref, bg_ref, gg_ref, bgg_ref, Wt_ref, bt_ref,
               W1_ref, b1_ref, g1_ref, bb1_ref, W2_ref, b2_ref,
               Wr1_ref, br1_ref, Wr2_ref, br2_ref, out_ref):
    cnt = cnt_ref[...]
    hsum = sum_ref[...]
    hmean = hsum / jnp.maximum(cnt, 1.0)
    hmax = jnp.where(cnt > 0, max_ref[...], 0.0)
    g = _ln(jnp.maximum(jnp.dot(gf_ref[...], Wg_ref[...],
                                preferred_element_type=jnp.float32) + bg_ref[...], 0.0),
            gg_ref[...], bgg_ref[...])
    toh = (tcls_ref[...] == lax.broadcasted_iota(jnp.int32, (B, 16), 1)).astype(jnp.float32)
    te = jnp.dot(toh, Temb_ref[...], preferred_element_type=jnp.float32)
    temb = jnp.maximum(jnp.dot(te, Wt_ref[...],
                               preferred_element_type=jnp.float32) + bt_ref[...], 0.0)
    c = jnp.concatenate([hmean, hmax, hsum, g, temb], axis=-1)
    c = _ln(jnp.maximum(jnp.dot(c, W1_ref[...],
                                preferred_element_type=jnp.float32) + b1_ref[...], 0.0),
            g1_ref[...], bb1_ref[...])
    c = jnp.maximum(jnp.dot(c, W2_ref[...],
                            preferred_element_type=jnp.float32) + b2_ref[...], 0.0)
    r = jnp.maximum(jnp.dot(c, Wr1_ref[...],
                            preferred_element_type=jnp.float32) + br1_ref[...], 0.0)
    out = jnp.dot(r, Wr2_ref[...], preferred_element_type=jnp.float32) + br2_ref[...]
    out_ref[...] = jnp.broadcast_to(out, (B, 128))


_head_call = pl.pallas_call(
    _head_body,
    out_shape=jax.ShapeDtypeStruct((B, 128), jnp.float32),
)


def kernel(x, edge_index, edge_attr, edge_gate_type, batch, global_features,
           threshold_class, W0, b0, g0, be0, mpWmsg, mpbmsg, mpWedge, mpGemb,
           mpWout, mpbout, mpWself, Wg, bg, gg, bgg, Temb, Wt, bt, W1, b1,
           g1, bb1, W2, b2, Wr1, br1, Wr2, br2):
    src = edge_index[0]
    dst = edge_index[1]
    gate3 = edge_gate_type.reshape(E // EBLK, EBLK, 1)
    batch3 = batch.reshape(N // NBLK, NBLK, 1)
    r1 = lambda a: a.reshape(1, -1)
    zero_chunk = jnp.zeros((ZCH, FW), jnp.float32)
    Temb_pad = jnp.pad(Temb, ((0, 16 - T), (0, 0)))

    h, hm = _init_call(x, W0, r1(b0), r1(g0), r1(be0), mpWmsg[0])
    e_all = _econst_call(edge_attr, gate3, mpWedge, mpGemb,
                         mpbmsg.reshape(L, 1, H))
    e_flat = e_all.reshape(L * FG * E, FW)
    for l in range(L):
        agg = _SC_CALLS[l](hm.reshape(FG * N, FW), e_flat, src, dst, zero_chunk)
        h, hm = _update_call(h, agg.reshape(FG, NPAD, FW), mpWout[l], mpWself[l],
                             r1(mpbout[l]), mpWmsg[(l + 1) % L])
    hsum, hmax, hcnt = _pool_call(h, batch3)
    out = _head_call(hsum, hmax, hcnt, global_features,
                     threshold_class.reshape(B, 1).astype(jnp.int32), Temb_pad,
                     Wg, r1(bg), r1(gg), r1(bgg), Wt, r1(bt),
                     W1, r1(b1), r1(g1), r1(bb1), W2, r1(b2),
                     Wr1, r1(br1), Wr2, r1(br2))
    return out[:, 0]


# speedup vs baseline: 2.8591x; 1.6398x over previous
import functools

import jax
import jax.numpy as jnp
from jax import lax
from jax.experimental import pallas as pl
from jax.experimental.pallas import tpu as pltpu
from jax.experimental.pallas import tpu_sc as plsc

N = 100000
E = 1600000
NF = 16
EF = 4
GF = 36
H = 64
L = 4
G = 8
B = 64
T = 9

FG = 4
FW = 16
NBLK = 1000
NTILES = 16
EPAD = 1638400
E8 = EPAD // 8
EPT = EPAD // NTILES
EBLK = 512
W = 256
WR = W // 8
NWIN = EPT // W
NPAD = 100096
NPT = NPAD // NTILES
ZCH = 184
NCH = NPT // ZCH


def _ln(v, g, b):
    m = jnp.mean(v, axis=-1, keepdims=True)
    var = jnp.mean((v - m) * (v - m), axis=-1, keepdims=True)
    return (v - m) * lax.rsqrt(var + 1e-5) * g + b


def _init_body(x_ref, W0_ref, b0_ref, g0_ref, be0_ref, Wm_ref, h_ref, hm_ref):
    h = jnp.maximum(jnp.dot(x_ref[...], W0_ref[...],
                            preferred_element_type=jnp.float32) + b0_ref[...], 0.0)
    h = _ln(h, g0_ref[...], be0_ref[...])
    h_ref[...] = h
    hm = jnp.dot(h, Wm_ref[...], preferred_element_type=jnp.float32)
    for fg in range(FG):
        hm_ref[fg] = hm[:, fg * FW:(fg + 1) * FW]


_init_call = pl.pallas_call(
    _init_body,
    grid=(N // NBLK,),
    in_specs=[
        pl.BlockSpec((NBLK, NF), lambda i: (i, 0)),
        pl.BlockSpec((NF, H), lambda i: (0, 0)),
        pl.BlockSpec((1, H), lambda i: (0, 0)),
        pl.BlockSpec((1, H), lambda i: (0, 0)),
        pl.BlockSpec((1, H), lambda i: (0, 0)),
        pl.BlockSpec((H, H), lambda i: (0, 0)),
    ],
    out_specs=[
        pl.BlockSpec((NBLK, H), lambda i: (i, 0)),
        pl.BlockSpec((FG, NBLK, FW), lambda i: (0, i, 0)),
    ],
    out_shape=[
        jax.ShapeDtypeStruct((N, H), jnp.float32),
        jax.ShapeDtypeStruct((FG, N, FW), jnp.float32),
    ],
)


def _econst_body(eaT_ref, gateT_ref, We_ref, Ge_ref, bm_ref, out_ref):
    eaT = eaT_ref[...]
    gateT = gateT_ref[...]
    ioh = lax.broadcasted_iota(jnp.int32, (EBLK, G), 1)
    for c in range(8):
        oh = (gateT[:, c:c + 1] == ioh).astype(jnp.float32)
        m = (jnp.dot(eaT[:, 4 * c:4 * c + 4], We_ref[0],
                     preferred_element_type=jnp.float32)
             + jnp.dot(oh, Ge_ref[0], preferred_element_type=jnp.float32)
             + bm_ref[0])
        for fg in range(FG):
            out_ref[0, fg, :, FW * c:FW * (c + 1)] = m[:, fg * FW:(fg + 1) * FW]


_econst_call = pl.pallas_call(
    _econst_body,
    grid=(L, E8 // EBLK),
    in_specs=[
        pl.BlockSpec((EBLK, 32), lambda l, e: (e, 0)),
        pl.BlockSpec((EBLK, G), lambda l, e: (e, 0)),
        pl.BlockSpec((1, EF, H), lambda l, e: (l, 0, 0)),
        pl.BlockSpec((1, G, H), lambda l, e: (l, 0, 0)),
        pl.BlockSpec((1, 1, H), lambda l, e: (l, 0, 0)),
    ],
    out_specs=pl.BlockSpec((1, FG, EBLK, 128), lambda l, e: (l, 0, e, 0)),
    out_shape=jax.ShapeDtypeStruct((L, FG, E8, 128), jnp.float32),
)


def _update_body(h_ref, agg_ref, Wout_ref, Wself_ref, bout_ref, Wm_ref,
                 hout_ref, hm_ref):
    h = h_ref[...]
    acc = jnp.dot(h, Wself_ref[...], preferred_element_type=jnp.float32) + bout_ref[...]
    for fg in range(FG):
        acc = acc + jnp.dot(agg_ref[fg], Wout_ref[...][fg * FW:(fg + 1) * FW, :],
                            preferred_element_type=jnp.float32)
    h2 = h + jnp.maximum(acc, 0.0)
    hout_ref[...] = h2
    hm = jnp.dot(h2, Wm_ref[...], preferred_element_type=jnp.float32)
    for fg in range(FG):
        hm_ref[fg] = hm[:, fg * FW:(fg + 1) * FW]


_update_call = pl.pallas_call(
    _update_body,
    grid=(N // NBLK,),
    in_specs=[
        pl.BlockSpec((NBLK, H), lambda i: (i, 0)),
        pl.BlockSpec((FG, NBLK, FW), lambda i: (0, i, 0)),
        pl.BlockSpec((H, H), lambda i: (0, 0)),
        pl.BlockSpec((H, H), lambda i: (0, 0)),
        pl.BlockSpec((1, H), lambda i: (0, 0)),
        pl.BlockSpec((H, H), lambda i: (0, 0)),
    ],
    out_specs=[
        pl.BlockSpec((NBLK, H), lambda i: (i, 0)),
        pl.BlockSpec((FG, NBLK, FW), lambda i: (0, i, 0)),
    ],
    out_shape=[
        jax.ShapeDtypeStruct((N, H), jnp.float32),
        jax.ShapeDtypeStruct((FG, N, FW), jnp.float32),
    ],
)


def _make_sc(layer):
    mesh = plsc.VectorSubcoreMesh(core_axis_name="c", subcore_axis_name="s")

    @functools.partial(
        pl.kernel,
        mesh=mesh,
        out_type=jax.ShapeDtypeStruct((FG * NPAD, FW), jnp.float32),
        compiler_params=pltpu.CompilerParams(use_tc_tiling_on_sc=False),
        scratch_types=[
            pltpu.VMEM((W,), jnp.int32),
            pltpu.VMEM((W,), jnp.int32),
            pltpu.VMEM((WR, 128), jnp.float32),
            pltpu.VMEM((W, FW), jnp.float32),
            pltpu.VMEM((W,), jnp.int32),
            pltpu.VMEM((W,), jnp.int32),
            pltpu.VMEM((WR, 128), jnp.float32),
            pltpu.VMEM((W, FW), jnp.float32),
            pltpu.VMEM_SHARED((NPAD, FW), jnp.float32),
            pltpu.SemaphoreType.DMA,
            pltpu.SemaphoreType.DMA,
            pltpu.SemaphoreType.DMA,
            pltpu.SemaphoreType.DMA,
            pltpu.SemaphoreType.DMA,
            pltpu.SemaphoreType.DMA,
        ],
    )
    def sc_kernel(hm_hbm, e_hbm, src_hbm, dst_hbm, zero_hbm, out_hbm,
                  srcv0, dstv0, elv0, hrv0, srcv1, dstv1, elv1, hrv1, aggs,
                  insem0, insem1, gsem0, gsem1, ssem0, ssem1):
        cid = lax.axis_index("c")
        sid = lax.axis_index("s")
        slots = ((srcv0, dstv0, elv0, hrv0, insem0, gsem0, ssem0),
                 (srcv1, dstv1, elv1, hrv1, insem1, gsem1, ssem1))
        for fgi in range(2):
            fg = fgi * 2 + cid
            pltpu.sync_copy(zero_hbm, hrv0.at[pl.ds(0, ZCH)])

            def zbody(k, _):
                off = pl.multiple_of(sid * NPT + k * ZCH, 8)
                pltpu.sync_copy(hrv0.at[pl.ds(0, ZCH)], aggs.at[pl.ds(off, ZCH)])
                return 0
            lax.fori_loop(0, NCH, zbody, 0)
            plsc.subcore_barrier()

            tile_lo = sid * EPT
            eoff = (layer * FG + fg) * E8 + sid * (EPT // 8)
            fgN = fg * N

            def _in_args(w, s):
                base = pl.multiple_of(tile_lo + w * W, 8)
                ebase = pl.multiple_of(eoff + w * WR, 8)
                return (
                    (src_hbm.at[pl.ds(base, W)], s[0], s[4]),
                    (dst_hbm.at[pl.ds(base, W)], s[1], s[4]),
                    (e_hbm.at[pl.ds(ebase, WR)], s[2], s[4]),
                )

            def inputs_start(w, s):
                for a in _in_args(w, s):
                    pltpu.async_copy(*a)

            def inputs_wait(w, s):
                for a in _in_args(w, s):
                    pltpu.make_async_copy(*a).wait()

            def idx_compute(s):
                def ib(j, _):
                    s[0][pl.ds(j * 16, 16)] = s[0][pl.ds(j * 16, 16)] + fgN
                    return 0
                lax.fori_loop(0, W // 16, ib, 0)

            def gather_start(s):
                pltpu.async_copy(hm_hbm.at[s[0]], s[3], s[5])

            def gather_wait(s):
                pltpu.make_async_copy(hm_hbm.at[s[0]], s[3], s[5]).wait()

            def msg_compute(s):
                ev, hv = s[2], s[3]

                def mb(r, _):
                    r8 = r * 8
                    for c in range(8):
                        hv[r8 + c] = jnp.maximum(
                            ev[r, pl.ds(FW * c, FW)] + hv[r8 + c], 0.0)
                    return 0
                lax.fori_loop(0, WR, mb, 0)

            def scatter_start(s):
                pltpu.async_copy(s[3], aggs.at[s[1]], s[6], add=True)

            def scatter_wait(s):
                pltpu.make_async_copy(s[3], aggs.at[s[1]], s[6]).wait()

            inputs_start(0, slots[0])
            inputs_wait(0, slots[0])
            idx_compute(slots[0])
            gather_start(slots[0])
            inputs_start(1, slots[1])

            def pair(k, _):
                w1 = 2 * k + 1
                w2 = 2 * k + 2
                w3 = 2 * k + 3
                s0, s1 = slots
                gather_wait(s0)
                msg_compute(s0)
                scatter_start(s0)
                inputs_wait(w1, s1)
                idx_compute(s1)
                gather_start(s1)
                scatter_wait(s0)

                @pl.when(w2 < NWIN)
                def _():
                    inputs_start(w2, s0)
                gather_wait(s1)
                msg_compute(s1)
                scatter_start(s1)

                @pl.when(w2 < NWIN)
                def _():
                    inputs_wait(w2, s0)
                    idx_compute(s0)
                    gather_start(s0)
                scatter_wait(s1)

                @pl.when(w3 < NWIN)
                def _():
                    inputs_start(w3, s1)
                return 0
            lax.fori_loop(0, NWIN // 2, pair, 0)
            plsc.subcore_barrier()

            def dbody(k, _):
                off = pl.multiple_of(sid * NPT + k * ZCH, 8)
                oout = pl.multiple_of(fg * NPAD + sid * NPT + k * ZCH, 8)
                pltpu.sync_copy(aggs.at[pl.ds(off, ZCH)], hrv0.at[pl.ds(0, ZCH)])
                pltpu.sync_copy(hrv0.at[pl.ds(0, ZCH)], out_hbm.at[pl.ds(oout, ZCH)])
                return 0
            lax.fori_loop(0, NCH, dbody, 0)
            plsc.subcore_barrier()

    return sc_kernel


_SC_CALLS = [_make_sc(l) for l in range(L)]


def _pool_body(h_ref, batch_ref, sum_ref, max_ref, cnt_ref):
    i = pl.program_id(0)

    @pl.when(i == 0)
    def _():
        sum_ref[...] = jnp.zeros_like(sum_ref)
        cnt_ref[...] = jnp.zeros_like(cnt_ref)
        max_ref[...] = jnp.full_like(max_ref, -jnp.inf)

    h = h_ref[...]
    bid = batch_ref[0]
    oh = (bid == lax.broadcasted_iota(jnp.int32, (NBLK, B), 1)).astype(jnp.float32)
    dn = (((0,), (0,)), ((), ()))
    sum_ref[...] += lax.dot_general(oh, h, dn, preferred_element_type=jnp.float32)
    cnt_ref[...] += lax.dot_general(oh, jnp.ones_like(h), dn,
                                    preferred_element_type=jnp.float32)
    lo = bid[0, 0]
    hi = bid[NBLK - 1, 0]
    for b in range(B):
        @pl.when((b >= lo) & (b <= hi))
        def _():
            mb = jnp.max(jnp.where(bid == b, h, -jnp.inf), axis=0)
            max_ref[b:b + 1, :] = jnp.maximum(max_ref[b:b + 1, :], mb[None, :])


_pool_call = pl.pallas_call(
    _pool_body,
    grid=(N // NBLK,),
    in_specs=[
        pl.BlockSpec((NBLK, H), lambda i: (i, 0)),
        pl.BlockSpec((1, NBLK, 1), lambda i: (i, 0, 0)),
    ],
    out_specs=[
        pl.BlockSpec((B, H), lambda i: (0, 0)),
        pl.BlockSpec((B, H), lambda i: (0, 0)),
        pl.BlockSpec((B, H), lambda i: (0, 0)),
    ],
    out_shape=[
        jax.ShapeDtypeStruct((B, H), jnp.float32),
        jax.ShapeDtypeStruct((B, H), jnp.float32),
        jax.ShapeDtypeStruct((B, H), jnp.float32),
    ],
)


def _head_body(sum_ref, max_ref, cnt_ref, gf_ref, tcls_ref, Temb_ref,
               Wg_ref, bg_ref, gg_ref, bgg_ref, Wt_ref, bt_ref,
               W1_ref, b1_ref, g1_ref, bb1_ref, W2_ref, b2_ref,
               Wr1_ref, br1_ref, Wr2_ref, br2_ref, out_ref):
    cnt = cnt_ref[...]
    hsum = sum_ref[...]
    hmean = hsum / jnp.maximum(cnt, 1.0)
    hmax = jnp.where(cnt > 0, max_ref[...], 0.0)
    g = _ln(jnp.maximum(jnp.dot(gf_ref[...], Wg_ref[...],
                                preferred_element_type=jnp.float32) + bg_ref[...], 0.0),
            gg_ref[...], bgg_ref[...])
    toh = (tcls_ref[...] == lax.broadcasted_iota(jnp.int32, (B, 16), 1)).astype(jnp.float32)
    te = jnp.dot(toh, Temb_ref[...], preferred_element_type=jnp.float32)
    temb = jnp.maximum(jnp.dot(te, Wt_ref[...],
                               preferred_element_type=jnp.float32) + bt_ref[...], 0.0)
    c = jnp.concatenate([hmean, hmax, hsum, g, temb], axis=-1)
    c = _ln(jnp.maximum(jnp.dot(c, W1_ref[...],
                                preferred_element_type=jnp.float32) + b1_ref[...], 0.0),
            g1_ref[...], bb1_ref[...])
    c = jnp.maximum(jnp.dot(c, W2_ref[...],
                            preferred_element_type=jnp.float32) + b2_ref[...], 0.0)
    r = jnp.maximum(jnp.dot(c, Wr1_ref[...],
                            preferred_element_type=jnp.float32) + br1_ref[...], 0.0)
    out = jnp.dot(r, Wr2_ref[...], preferred_element_type=jnp.float32) + br2_ref[...]
    out_ref[...] = jnp.broadcast_to(out, (B, 128))


_head_call = pl.pallas_call(
    _head_body,
    out_shape=jax.ShapeDtypeStruct((B, 128), jnp.float32),
)


def kernel(x, edge_index, edge_attr, edge_gate_type, batch, global_features,
           threshold_class, W0, b0, g0, be0, mpWmsg, mpbmsg, mpWedge, mpGemb,
           mpWout, mpbout, mpWself, Wg, bg, gg, bgg, Temb, Wt, bt, W1, b1,
           g1, bb1, W2, b2, Wr1, br1, Wr2, br2):
    src = edge_index[0]
    dst = edge_index[1]
    batch3 = batch.reshape(N // NBLK, NBLK, 1)
    r1 = lambda a: a.reshape(1, -1)
    zero_chunk = jnp.zeros((ZCH, FW), jnp.float32)
    Temb_pad = jnp.pad(Temb, ((0, 16 - T), (0, 0)))

    pe = EPAD - E
    pidx = jnp.arange(pe, dtype=jnp.int32)
    src_pad = jnp.concatenate([src, pidx % N])
    dst_pad = jnp.concatenate([dst, N + pidx % (NPAD - N)])
    ea_pad = jnp.concatenate([edge_attr, jnp.zeros((pe, EF), jnp.float32)])
    gate_pad = jnp.concatenate([edge_gate_type, jnp.zeros((pe,), jnp.int32)])
    eaT = ea_pad.reshape(8, E8, EF).transpose(1, 0, 2).reshape(E8, 8 * EF)
    gateT = gate_pad.reshape(8, E8).transpose(1, 0)
    srcp = src_pad.reshape(8, E8).transpose(1, 0).reshape(EPAD)
    dstp = dst_pad.reshape(8, E8).transpose(1, 0).reshape(EPAD)

    h, hm = _init_call(x, W0, r1(b0), r1(g0), r1(be0), mpWmsg[0])
    e_all = _econst_call(eaT, gateT, mpWedge, mpGemb, mpbmsg.reshape(L, 1, H))
    e_flat = e_all.reshape(L * FG * E8, 128)
    for l in range(L):
        agg = _SC_CALLS[l](hm.reshape(FG * N, FW), e_flat, srcp, dstp, zero_chunk)
        h, hm = _update_call(h, agg.reshape(FG, NPAD, FW), mpWout[l], mpWself[l],
                             r1(mpbout[l]), mpWmsg[(l + 1) % L])
    hsum, hmax, hcnt = _pool_call(h, batch3)
    out = _head_call(hsum, hmax, hcnt, global_features,
                     threshold_class.reshape(B, 1).astype(jnp.int32), Temb_pad,
                     Wg, r1(bg), r1(gg), r1(bgg), Wt, r1(bt),
                     W1, r1(b1), r1(g1), r1(bb1), W2, r1(b2),
                     Wr1, r1(br1), Wr2, r1(br2))
    return out[:, 0]


# speedup vs baseline: 4.4927x; 1.5713x over previous
import functools

import jax
import jax.numpy as jnp
from jax import lax
from jax.experimental import pallas as pl
from jax.experimental.pallas import tpu as pltpu
from jax.experimental.pallas import tpu_sc as plsc

N = 100000
E = 1600000
NF = 16
EF = 4
GF = 36
H = 64
L = 4
G = 8
B = 64
T = 9

FG = 4
FW = 16
NBLK = 1000
NTILES = 16
EPAD = 1638400
E8 = EPAD // 8
EPT = EPAD // NTILES
EBLK = 512
W = 256
WR = W // 8
NWIN = EPT // W
NPAD = 100096
NPT = NPAD // NTILES
ZCH = 184
NCH = NPT // ZCH


def _ln(v, g, b):
    m = jnp.mean(v, axis=-1, keepdims=True)
    var = jnp.mean((v - m) * (v - m), axis=-1, keepdims=True)
    return (v - m) * lax.rsqrt(var + 1e-5) * g + b


def _init_body(x_ref, W0_ref, b0_ref, g0_ref, be0_ref, Wm_ref, h_ref, hm_ref):
    h = jnp.maximum(jnp.dot(x_ref[...], W0_ref[...],
                            preferred_element_type=jnp.float32) + b0_ref[...], 0.0)
    h = _ln(h, g0_ref[...], be0_ref[...])
    h_ref[...] = h
    hm = jnp.dot(h, Wm_ref[...], preferred_element_type=jnp.float32)
    for fg in range(FG):
        hm_ref[fg] = hm[:, fg * FW:(fg + 1) * FW]


_init_call = pl.pallas_call(
    _init_body,
    grid=(N // NBLK,),
    in_specs=[
        pl.BlockSpec((NBLK, NF), lambda i: (i, 0)),
        pl.BlockSpec((NF, H), lambda i: (0, 0)),
        pl.BlockSpec((1, H), lambda i: (0, 0)),
        pl.BlockSpec((1, H), lambda i: (0, 0)),
        pl.BlockSpec((1, H), lambda i: (0, 0)),
        pl.BlockSpec((H, H), lambda i: (0, 0)),
    ],
    out_specs=[
        pl.BlockSpec((NBLK, H), lambda i: (i, 0)),
        pl.BlockSpec((FG, NBLK, FW), lambda i: (0, i, 0)),
    ],
    out_shape=[
        jax.ShapeDtypeStruct((N, H), jnp.float32),
        jax.ShapeDtypeStruct((FG, N, FW), jnp.float32),
    ],
)


def _econst_body(aug_ref, Wf_ref, bT_ref, out_ref):
    aug = aug_ref[...]
    for fg in range(FG):
        out_ref[0, fg] = (jnp.dot(aug, Wf_ref[0, fg],
                                  preferred_element_type=jnp.float32)
                          + bT_ref[0, fg])


_econst_call = pl.pallas_call(
    _econst_body,
    grid=(L, E8 // EBLK),
    in_specs=[
        pl.BlockSpec((EBLK, 96), lambda l, e: (e, 0)),
        pl.BlockSpec((1, FG, 96, 128), lambda l, e: (l, 0, 0, 0)),
        pl.BlockSpec((1, FG, 1, 128), lambda l, e: (l, 0, 0, 0)),
    ],
    out_specs=pl.BlockSpec((1, FG, EBLK, 128), lambda l, e: (l, 0, e, 0)),
    out_shape=jax.ShapeDtypeStruct((L, FG, E8, 128), jnp.float32),
)


def _update_body(h_ref, agg_ref, Wout_ref, Wself_ref, bout_ref, Wm_ref,
                 hout_ref, hm_ref):
    h = h_ref[...]
    acc = jnp.dot(h, Wself_ref[...], preferred_element_type=jnp.float32) + bout_ref[...]
    for fg in range(FG):
        acc = acc + jnp.dot(agg_ref[fg], Wout_ref[...][fg * FW:(fg + 1) * FW, :],
                            preferred_element_type=jnp.float32)
    h2 = h + jnp.maximum(acc, 0.0)
    hout_ref[...] = h2
    hm = jnp.dot(h2, Wm_ref[...], preferred_element_type=jnp.float32)
    for fg in range(FG):
        hm_ref[fg] = hm[:, fg * FW:(fg + 1) * FW]


_update_call = pl.pallas_call(
    _update_body,
    grid=(N // NBLK,),
    in_specs=[
        pl.BlockSpec((NBLK, H), lambda i: (i, 0)),
        pl.BlockSpec((FG, NBLK, FW), lambda i: (0, i, 0)),
        pl.BlockSpec((H, H), lambda i: (0, 0)),
        pl.BlockSpec((H, H), lambda i: (0, 0)),
        pl.BlockSpec((1, H), lambda i: (0, 0)),
        pl.BlockSpec((H, H), lambda i: (0, 0)),
    ],
    out_specs=[
        pl.BlockSpec((NBLK, H), lambda i: (i, 0)),
        pl.BlockSpec((FG, NBLK, FW), lambda i: (0, i, 0)),
    ],
    out_shape=[
        jax.ShapeDtypeStruct((N, H), jnp.float32),
        jax.ShapeDtypeStruct((FG, N, FW), jnp.float32),
    ],
)


def _make_sc(layer):
    mesh = plsc.VectorSubcoreMesh(core_axis_name="c", subcore_axis_name="s")

    @functools.partial(
        pl.kernel,
        mesh=mesh,
        out_type=jax.ShapeDtypeStruct((FG * NPAD, FW), jnp.float32),
        compiler_params=pltpu.CompilerParams(use_tc_tiling_on_sc=False),
        scratch_types=[
            pltpu.VMEM((W,), jnp.int32),
            pltpu.VMEM((W,), jnp.int32),
            pltpu.VMEM((WR, 128), jnp.float32),
            pltpu.VMEM((W, FW), jnp.float32),
            pltpu.VMEM((W,), jnp.int32),
            pltpu.VMEM((W,), jnp.int32),
            pltpu.VMEM((WR, 128), jnp.float32),
            pltpu.VMEM((W, FW), jnp.float32),
            pltpu.VMEM_SHARED((NPAD, FW), jnp.float32),
            pltpu.SemaphoreType.DMA,
            pltpu.SemaphoreType.DMA,
            pltpu.SemaphoreType.DMA,
            pltpu.SemaphoreType.DMA,
            pltpu.SemaphoreType.DMA,
            pltpu.SemaphoreType.DMA,
        ],
    )
    def sc_kernel(hm_hbm, e_hbm, src_hbm, dst_hbm, zero_hbm, out_hbm,
                  srcv0, dstv0, elv0, hrv0, srcv1, dstv1, elv1, hrv1, aggs,
                  insem0, insem1, gsem0, gsem1, ssem0, ssem1):
        cid = lax.axis_index("c")
        sid = lax.axis_index("s")
        slots = ((srcv0, dstv0, elv0, hrv0, insem0, gsem0, ssem0),
                 (srcv1, dstv1, elv1, hrv1, insem1, gsem1, ssem1))
        for fgi in range(2):
            fg = fgi * 2 + cid
            pltpu.sync_copy(zero_hbm, hrv0.at[pl.ds(0, ZCH)])

            def zbody(k, _):
                off = pl.multiple_of(sid * NPT + k * ZCH, 8)
                pltpu.sync_copy(hrv0.at[pl.ds(0, ZCH)], aggs.at[pl.ds(off, ZCH)])
                return 0
            lax.fori_loop(0, NCH, zbody, 0)
            plsc.subcore_barrier()

            tile_lo = sid * EPT
            eoff = (layer * FG + fg) * E8 + sid * (EPT // 8)
            fgN = fg * N

            def _in_args(w, s):
                base = pl.multiple_of(tile_lo + w * W, 8)
                ebase = pl.multiple_of(eoff + w * WR, 8)
                return (
                    (src_hbm.at[pl.ds(base, W)], s[0], s[4]),
                    (dst_hbm.at[pl.ds(base, W)], s[1], s[4]),
                    (e_hbm.at[pl.ds(ebase, WR)], s[2], s[4]),
                )

            def inputs_start(w, s):
                for a in _in_args(w, s):
                    pltpu.async_copy(*a)

            def inputs_wait(w, s):
                for a in _in_args(w, s):
                    pltpu.make_async_copy(*a).wait()

            def idx_compute(s):
                def ib(j, _):
                    s[0][pl.ds(j * 16, 16)] = s[0][pl.ds(j * 16, 16)] + fgN
                    return 0
                lax.fori_loop(0, W // 16, ib, 0)

            def gather_start(s):
                pltpu.async_copy(hm_hbm.at[s[0]], s[3], s[5])

            def gather_wait(s):
                pltpu.make_async_copy(hm_hbm.at[s[0]], s[3], s[5]).wait()

            def msg_compute(s):
                ev, hv = s[2], s[3]

                def mb(r, _):
                    r8 = r * 8
                    for c in range(8):
                        hv[r8 + c] = jnp.maximum(
                            ev[r, pl.ds(FW * c, FW)] + hv[r8 + c], 0.0)
                    return 0
                lax.fori_loop(0, WR, mb, 0)

            def scatter_start(s):
                pltpu.async_copy(s[3], aggs.at[s[1]], s[6], add=True)

            def scatter_wait(s):
                pltpu.make_async_copy(s[3], aggs.at[s[1]], s[6]).wait()

            inputs_start(0, slots[0])
            inputs_wait(0, slots[0])
            idx_compute(slots[0])
            gather_start(slots[0])
            inputs_start(1, slots[1])

            def pair(k, _):
                w1 = 2 * k + 1
                w2 = 2 * k + 2
                w3 = 2 * k + 3
                s0, s1 = slots
                gather_wait(s0)
                msg_compute(s0)
                scatter_start(s0)
                inputs_wait(w1, s1)
                idx_compute(s1)
                gather_start(s1)
                scatter_wait(s0)

                @pl.when(w2 < NWIN)
                def _():
                    inputs_start(w2, s0)
                gather_wait(s1)
                msg_compute(s1)
                scatter_start(s1)

                @pl.when(w2 < NWIN)
                def _():
                    inputs_wait(w2, s0)
                    idx_compute(s0)
                    gather_start(s0)
                scatter_wait(s1)

                @pl.when(w3 < NWIN)
                def _():
                    inputs_start(w3, s1)
                return 0
            lax.fori_loop(0, NWIN // 2, pair, 0)
            plsc.subcore_barrier()

            def dbody(k, _):
                off = pl.multiple_of(sid * NPT + k * ZCH, 8)
                oout = pl.multiple_of(fg * NPAD + sid * NPT + k * ZCH, 8)
                pltpu.sync_copy(aggs.at[pl.ds(off, ZCH)], hrv0.at[pl.ds(0, ZCH)])
                pltpu.sync_copy(hrv0.at[pl.ds(0, ZCH)], out_hbm.at[pl.ds(oout, ZCH)])
                return 0
            lax.fori_loop(0, NCH, dbody, 0)
            plsc.subcore_barrier()

    return sc_kernel


_SC_CALLS = [_make_sc(l) for l in range(L)]


def _pool_body(h_ref, batch_ref, sum_ref, max_ref, cnt_ref):
    i = pl.program_id(0)

    @pl.when(i == 0)
    def _():
        sum_ref[...] = jnp.zeros_like(sum_ref)
        cnt_ref[...] = jnp.zeros_like(cnt_ref)
        max_ref[...] = jnp.full_like(max_ref, -jnp.inf)

    h = h_ref[...]
    bid = batch_ref[0]
    oh = (bid == lax.broadcasted_iota(jnp.int32, (NBLK, B), 1)).astype(jnp.float32)
    dn = (((0,), (0,)), ((), ()))
    sum_ref[...] += lax.dot_general(oh, h, dn, preferred_element_type=jnp.float32)
    cnt_ref[...] += lax.dot_general(oh, jnp.ones_like(h), dn,
                                    preferred_element_type=jnp.float32)
    lo = bid[0, 0]
    hi = bid[NBLK - 1, 0]
    for b in range(B):
        @pl.when((b >= lo) & (b <= hi))
        def _():
            mb = jnp.max(jnp.where(bid == b, h, -jnp.inf), axis=0)
            max_ref[b:b + 1, :] = jnp.maximum(max_ref[b:b + 1, :], mb[None, :])


_pool_call = pl.pallas_call(
    _pool_body,
    grid=(N // NBLK,),
    in_specs=[
        pl.BlockSpec((NBLK, H), lambda i: (i, 0)),
        pl.BlockSpec((1, NBLK, 1), lambda i: (i, 0, 0)),
    ],
    out_specs=[
        pl.BlockSpec((B, H), lambda i: (0, 0)),
        pl.BlockSpec((B, H), lambda i: (0, 0)),
        pl.BlockSpec((B, H), lambda i: (0, 0)),
    ],
    out_shape=[
        jax.ShapeDtypeStruct((B, H), jnp.float32),
        jax.ShapeDtypeStruct((B, H), jnp.float32),
        jax.ShapeDtypeStruct((B, H), jnp.float32),
    ],
)


def _head_body(sum_ref, max_ref, cnt_ref, gf_ref, tcls_ref, Temb_ref,
               Wg_ref, bg_ref, gg_ref, bgg_ref, Wt_ref, bt_ref,
               W1_ref, b1_ref, g1_ref, bb1_ref, W2_ref, b2_ref,
               Wr1_ref, br1_ref, Wr2_ref, br2_ref, out_ref):
    cnt = cnt_ref[...]
    hsum = sum_ref[...]
    hmean = hsum / jnp.maximum(cnt, 1.0)
    hmax = jnp.where(cnt > 0, max_ref[...], 0.0)
    g = _ln(jnp.maximum(jnp.dot(gf_ref[...], Wg_ref[...],
                                preferred_element_type=jnp.float32) + bg_ref[...], 0.0),
            gg_ref[...], bgg_ref[...])
    toh = (tcls_ref[...] == lax.broadcasted_iota(jnp.int32, (B, 16), 1)).astype(jnp.float32)
    te = jnp.dot(toh, Temb_ref[...], preferred_element_type=jnp.float32)
    temb = jnp.maximum(jnp.dot(te, Wt_ref[...],
                               preferred_element_type=jnp.float32) + bt_ref[...], 0.0)
    c = jnp.concatenate([hmean, hmax, hsum, g, temb], axis=-1)
    c = _ln(jnp.maximum(jnp.dot(c, W1_ref[...],
                                preferred_element_type=jnp.float32) + b1_ref[...], 0.0),
            g1_ref[...], bb1_ref[...])
    c = jnp.maximum(jnp.dot(c, W2_ref[...],
                            preferred_element_type=jnp.float32) + b2_ref[...], 0.0)
    r = jnp.maximum(jnp.dot(c, Wr1_ref[...],
                            preferred_element_type=jnp.float32) + br1_ref[...], 0.0)
    out = jnp.dot(r, Wr2_ref[...], preferred_element_type=jnp.float32) + br2_ref[...]
    out_ref[...] = jnp.broadcast_to(out, (B, 128))


_head_call = pl.pallas_call(
    _head_body,
    out_shape=jax.ShapeDtypeStruct((B, 128), jnp.float32),
)


def kernel(x, edge_index, edge_attr, edge_gate_type, batch, global_features,
           threshold_class, W0, b0, g0, be0, mpWmsg, mpbmsg, mpWedge, mpGemb,
           mpWout, mpbout, mpWself, Wg, bg, gg, bgg, Temb, Wt, bt, W1, b1,
           g1, bb1, W2, b2, Wr1, br1, Wr2, br2):
    src = edge_index[0]
    dst = edge_index[1]
    batch3 = batch.reshape(N // NBLK, NBLK, 1)
    r1 = lambda a: a.reshape(1, -1)
    zero_chunk = jnp.zeros((ZCH, FW), jnp.float32)
    Temb_pad = jnp.pad(Temb, ((0, 16 - T), (0, 0)))

    pe = EPAD - E
    pidx = jnp.arange(pe, dtype=jnp.int32)
    src_pad = jnp.concatenate([src, pidx % N])
    dst_pad = jnp.concatenate([dst, N + pidx % (NPAD - N)])
    ea_pad = jnp.concatenate([edge_attr, jnp.zeros((pe, EF), jnp.float32)])
    gate_pad = jnp.concatenate([edge_gate_type, jnp.zeros((pe,), jnp.int32)])
    eaT = ea_pad.reshape(8, E8, EF).transpose(1, 0, 2).reshape(E8, 8 * EF)
    gateT = gate_pad.reshape(8, E8).transpose(1, 0)
    ohT = (gateT[:, :, None] == jnp.arange(G, dtype=jnp.int32)
           ).astype(jnp.float32).reshape(E8, 8 * G)
    aug = jnp.concatenate([eaT, ohT], axis=1)
    srcp = src_pad.reshape(8, E8).transpose(1, 0).reshape(EPAD)
    dstp = dst_pad.reshape(8, E8).transpose(1, 0).reshape(EPAD)

    I8 = jnp.eye(8, dtype=jnp.float32)
    Wf = jnp.stack([
        jnp.stack([
            jnp.concatenate([
                jnp.kron(I8, mpWedge[l][:, fg * FW:(fg + 1) * FW]),
                jnp.kron(I8, mpGemb[l][:, fg * FW:(fg + 1) * FW]),
            ], axis=0)
            for fg in range(FG)])
        for l in range(L)])
    bT = jnp.stack([
        jnp.stack([jnp.tile(mpbmsg[l, fg * FW:(fg + 1) * FW], 8)[None, :]
                   for fg in range(FG)])
        for l in range(L)])

    h, hm = _init_call(x, W0, r1(b0), r1(g0), r1(be0), mpWmsg[0])
    e_all = _econst_call(aug, Wf, bT)
    e_flat = e_all.reshape(L * FG * E8, 128)
    for l in range(L):
        agg = _SC_CALLS[l](hm.reshape(FG * N, FW), e_flat, srcp, dstp, zero_chunk)
        h, hm = _update_call(h, agg.reshape(FG, NPAD, FW), mpWout[l], mpWself[l],
                             r1(mpbout[l]), mpWmsg[(l + 1) % L])
    hsum, hmax, hcnt = _pool_call(h, batch3)
    out = _head_call(hsum, hmax, hcnt, global_features,
                     threshold_class.reshape(B, 1).astype(jnp.int32), Temb_pad,
                     Wg, r1(bg), r1(gg), r1(bgg), Wt, r1(bt),
                     W1, r1(b1), r1(g1), r1(bb1), W2, r1(b2),
                     Wr1, r1(br1), Wr2, r1(br2))
    return out[:, 0]
